# baseline (speedup 1.0000x reference)
import functools

import jax
import jax.numpy as jnp
from jax import lax
from jax.experimental import pallas as pl
from jax.experimental.pallas import tpu as pltpu
from jax.experimental.pallas import tpu_sc as plsc

N_NODES = 100000
N_EDGES = 3200000
D = 128

NC = 2
NS = 16
NW = NC * NS

R = 100352
NODES_PER_W = R // NW
VECS2 = NODES_PER_W // 16
SLICE1 = (2 * R) // NS

EPW = N_EDGES // NW
CHUNK = 10000
VECS1 = CHUNK // 16
NCHUNK = EPW // CHUNK
NBUF = 2
GU = 25
NGRP = VECS1 // GU


def _phase1(src_hbm, dst_hbm, iv_hbm, jv_hbm, masks_out,
            srcb0, srcb1, dstb, ivb, jvb, onesb, zerob, mask_sh, sem0, sem1):
    cid = lax.axis_index("c")
    sid = lax.axis_index("s")
    wid = cid * NS + sid

    iota = lax.iota(jnp.int32, 16)
    ebase = wid * EPW
    bufs = (srcb0, srcb1)
    sems = (sem0, sem1)

    def chunk_ref(ci):
        return src_hbm.at[pl.ds(ebase + ci * CHUNK, CHUNK)]

    pltpu.async_copy(chunk_ref(0), srcb0, sem0)
    pltpu.async_copy(chunk_ref(1), srcb1, sem1)

    pltpu.sync_copy(iv_hbm, ivb)
    pltpu.sync_copy(jv_hbm, jvb)
    onesb[...] = jnp.ones((16,), jnp.float32)
    zf = jnp.zeros((16,), jnp.float32)

    def zbody(k, _):
        zerob[pl.ds(k * 16, 16)] = zf
        return 0
    lax.fori_loop(0, SLICE1 // 16, zbody, 0)
    pltpu.sync_copy(zerob, mask_sh.at[pl.ds(sid * SLICE1, SLICE1)])
    plsc.subcore_barrier()

    iv = ivb[...]
    jv = jvb[...]
    dump_i = N_NODES + iota
    dump_j = R + N_NODES + iota
    zi = jnp.zeros((16,), jnp.int32)

    def scan_chunk(ci, buf, sem):
        pltpu.make_async_copy(chunk_ref(ci), buf, sem).wait()

        @plsc.parallel_loop(0, NGRP, carry=zi, unroll=2)
        def scan_group(g, acc):
            b0 = g * (GU * 16)
            ms = []
            for t in range(GU):
                v = buf[pl.ds(b0 + t * 16, 16)]
                ms.append((v == iv) | (v == jv))
            while len(ms) > 1:
                nxt = [ms[t] | ms[t + 1] for t in range(0, len(ms) - 1, 2)]
                if len(ms) % 2:
                    nxt.append(ms[-1])
                ms = nxt
            return acc | jnp.where(ms[0], 1, 0)
        anyv = scan_group
        anyn = jnp.max(plsc.all_reduce_population_count(anyv > 0))

        @pl.when(anyn > 0)
        def _slow():
            pltpu.sync_copy(dst_hbm.at[pl.ds(ebase + ci * CHUNK, CHUNK)], dstb)

            def gbody(g, _):
                b0 = g * (GU * 16)
                m16 = zi
                for t in range(GU):
                    v = buf[pl.ds(b0 + t * 16, 16)]
                    m16 = m16 | jnp.where((v == iv) | (v == jv), 1, 0)
                gn = jnp.max(plsc.all_reduce_population_count(m16 > 0))

                @pl.when(gn > 0)
                def _ghit():
                    for t in range(GU):
                        v = buf[pl.ds(b0 + t * 16, 16)]
                        ei = v == iv
                        ej = v == jv
                        m = jnp.max(plsc.all_reduce_population_count(ei | ej))

                        @pl.when(m > 0)
                        def _hit(t=t, ei=ei, ej=ej):
                            vd = dstb[pl.ds(b0 + t * 16, 16)]
                            idx_i = jnp.where(ei, vd, dump_i)
                            idx_j = jnp.where(ej, vd + R, dump_j)
                            pltpu.sync_copy(onesb, mask_sh.at[idx_i], add=True)
                            pltpu.sync_copy(onesb, mask_sh.at[idx_j], add=True)
                return 0
            lax.fori_loop(0, NGRP, gbody, 0)

    def outer(o, _):
        for b in range(NBUF):
            ci = o * NBUF + b
            scan_chunk(ci, bufs[b], sems[b])

            @pl.when(ci + NBUF < NCHUNK)
            def _next(ci=ci, b=b):
                pltpu.async_copy(chunk_ref(ci + NBUF), bufs[b], sems[b])
        return 0
    lax.fori_loop(0, NCHUNK // NBUF, outer, 0)

    plsc.subcore_barrier()
    off = sid * SLICE1
    pltpu.sync_copy(mask_sh.at[pl.ds(off, SLICE1)],
                    masks_out.at[pl.ds(cid * 2 * R + off, SLICE1)])


def _phase2(masks_hbm, emb_hbm, tl_hbm, partials_out, product_out,
            mi0b, mi1b, mj0b, mj1b, hitb, accb, rowb, tlb, ijrows, prodb, sem):
    cid = lax.axis_index("c")
    sid = lax.axis_index("s")
    wid = cid * NS + sid

    iota = lax.iota(jnp.int32, 16)
    base = wid * NODES_PER_W

    pltpu.sync_copy(masks_hbm.at[pl.ds(base, NODES_PER_W)], mi0b)
    pltpu.sync_copy(masks_hbm.at[pl.ds(2 * R + base, NODES_PER_W)], mi1b)
    pltpu.sync_copy(masks_hbm.at[pl.ds(R + base, NODES_PER_W)], mj0b)
    pltpu.sync_copy(masks_hbm.at[pl.ds(3 * R + base, NODES_PER_W)], mj1b)

    zf = jnp.zeros((16,), jnp.float32)
    for v in range(D // 16):
        accb[pl.ds(v * 16, 16)] = zf

    def common_at(k):
        s = pl.ds(k * 16, 16)
        mi = mi0b[s] + mi1b[s]
        mj = mj0b[s] + mj1b[s]
        u = base + k * 16 + iota
        return (mi > 0.0) & (mj > 0.0) & (u < N_NODES), u

    def qbody(k, acc):
        c, _ = common_at(k)
        return acc | jnp.where(c, 1, 0)
    anyv = lax.fori_loop(0, VECS2, qbody, jnp.zeros((16,), jnp.int32))
    anyn = jnp.max(plsc.all_reduce_population_count(anyv > 0))

    @pl.when(anyn > 0)
    def _collect():
        zi = jnp.zeros((16,), jnp.int32)

        def pbody(k, _):
            hitb[pl.ds(k * 16, 16)] = zi
            return 0
        lax.fori_loop(0, (NODES_PER_W + 16) // 16, pbody, 0)

        def cbody(k, off):
            c, u = common_at(k)
            plsc.store_compressed(hitb.at[pl.ds(off, 16)], u, mask=c)
            return off + jnp.max(plsc.all_reduce_population_count(c))
        nh = lax.fori_loop(0, VECS2, cbody, 0)

        ngr = (nh + 7) // 8

        def gbody(g, _):
            pltpu.sync_copy(emb_hbm.at[hitb.at[pl.ds(g * 8, 8)]], rowb)
            for r in range(8):
                for v in range(D // 16):
                    s = pl.ds(v * 16, 16)
                    accb[s] = accb[s] + rowb[r, s]
            return 0
        lax.fori_loop(0, ngr, gbody, 0)

        pad = (ngr * 8 - nh).astype(jnp.float32)

        @pl.when(pad > 0.0)
        def _unpad():
            padv = jnp.broadcast_to(pad, (16,))
            for v in range(D // 16):
                s = pl.ds(v * 16, 16)
                accb[s] = accb[s] - padv * rowb[7, s]

    pltpu.sync_copy(accb, partials_out.at[pl.ds(wid * D, D)])

    @pl.when(wid == 0)
    def _product():
        pltpu.sync_copy(tl_hbm, tlb)
        pltpu.sync_copy(emb_hbm.at[tlb.at[pl.ds(0, 2)]], ijrows)
        for v in range(D // 16):
            s = pl.ds(v * 16, 16)
            prodb[s] = ijrows[0, s] * ijrows[1, s]
        pltpu.sync_copy(prodb, product_out)


_mesh = plsc.VectorSubcoreMesh(core_axis_name="c", subcore_axis_name="s")
_params = pltpu.CompilerParams(needs_layout_passes=False)

_phase1_call = functools.partial(
    pl.kernel,
    out_type=jax.ShapeDtypeStruct((NC * 2 * R,), jnp.float32),
    mesh=_mesh,
    compiler_params=_params,
    scratch_types=[
        pltpu.VMEM((CHUNK,), jnp.int32),
        pltpu.VMEM((CHUNK,), jnp.int32),
        pltpu.VMEM((CHUNK,), jnp.int32),
        pltpu.VMEM((16,), jnp.int32),
        pltpu.VMEM((16,), jnp.int32),
        pltpu.VMEM((16,), jnp.float32),
        pltpu.VMEM((SLICE1,), jnp.float32),
        pltpu.VMEM_SHARED((2 * R,), jnp.float32),
        pltpu.SemaphoreType.DMA,
        pltpu.SemaphoreType.DMA,
    ],
)(_phase1)

_phase2_call = functools.partial(
    pl.kernel,
    out_type=(
        jax.ShapeDtypeStruct((NW * D,), jnp.float32),
        jax.ShapeDtypeStruct((D,), jnp.float32),
    ),
    mesh=_mesh,
    compiler_params=_params,
    scratch_types=[
        pltpu.VMEM((NODES_PER_W,), jnp.float32),
        pltpu.VMEM((NODES_PER_W,), jnp.float32),
        pltpu.VMEM((NODES_PER_W,), jnp.float32),
        pltpu.VMEM((NODES_PER_W,), jnp.float32),
        pltpu.VMEM((NODES_PER_W + 16,), jnp.int32),
        pltpu.VMEM((D,), jnp.float32),
        pltpu.VMEM((8, D), jnp.float32),
        pltpu.VMEM((16,), jnp.int32),
        pltpu.VMEM((2, D), jnp.float32),
        pltpu.VMEM((D,), jnp.float32),
        pltpu.SemaphoreType.DMA,
    ],
)(_phase2)


@jax.jit
def kernel(targetLink, adjacent, NodeEmbedding):
    src = adjacent[0]
    dst = adjacent[1]
    iv = jnp.broadcast_to(targetLink[0], (16,)).astype(jnp.int32)
    jv = jnp.broadcast_to(targetLink[1], (16,)).astype(jnp.int32)
    tl = jnp.zeros((16,), jnp.int32).at[:2].set(targetLink)
    masks = _phase1_call(src, dst, iv, jv)
    partials, product = _phase2_call(masks, NodeEmbedding, tl)
    return jnp.concatenate(
        [product, jnp.sum(partials.reshape(NW, D), axis=0)], axis=0)


# speedup vs baseline: 1.3840x; 1.3840x over previous
import functools

import jax
import jax.numpy as jnp
from jax import lax
from jax.experimental import pallas as pl
from jax.experimental.pallas import tpu as pltpu
from jax.experimental.pallas import tpu_sc as plsc

N_NODES = 100000
N_EDGES = 3200000
D = 128

NC = 2
NS = 16
NW = NC * NS

R = 100352
NODES_PER_W = R // NW
VECS2 = NODES_PER_W // 16
SLICE1 = (2 * R) // NS

EPW = N_EDGES // NW
CHUNK = 10000
VECS1 = CHUNK // 16
NCHUNK = EPW // CHUNK
NBUF = 2
GU = 25
NGRP = VECS1 // GU


def _phase1(src_hbm, dst_hbm, iv_hbm, jv_hbm, masks_out,
            srcb0, srcb1, dstb, ivb, jvb, onesb, zerob, mask_sh, sem0, sem1):
    cid = lax.axis_index("c")
    sid = lax.axis_index("s")
    wid = cid * NS + sid

    iota = lax.iota(jnp.int32, 16)
    ebase = wid * EPW
    bufs = (srcb0, srcb1)
    sems = (sem0, sem1)

    def chunk_ref(ci):
        return src_hbm.at[pl.ds(ebase + ci * CHUNK, CHUNK)]

    pltpu.async_copy(chunk_ref(0), srcb0, sem0)
    pltpu.async_copy(chunk_ref(1), srcb1, sem1)

    pltpu.sync_copy(iv_hbm, ivb)
    pltpu.sync_copy(jv_hbm, jvb)
    onesb[...] = jnp.ones((16,), jnp.float32)
    zf = jnp.zeros((16,), jnp.float32)

    def zbody(k, _):
        zerob[pl.ds(k * 16, 16)] = zf
        return 0
    lax.fori_loop(0, SLICE1 // 16, zbody, 0)
    pltpu.sync_copy(zerob, mask_sh.at[pl.ds(sid * SLICE1, SLICE1)])
    plsc.subcore_barrier()

    iv = ivb[...]
    jv = jvb[...]
    dump_i = N_NODES + iota
    dump_j = R + N_NODES + iota
    zi = jnp.zeros((16,), jnp.int32)

    def scan_chunk(ci, buf, sem):
        pltpu.make_async_copy(chunk_ref(ci), buf, sem).wait()

        def scan_group(g, acc):
            b0 = g * (GU * 16)
            ms = []
            for t in range(GU):
                v = buf[pl.ds(b0 + t * 16, 16)]
                ms.append((v == iv) | (v == jv))
            while len(ms) > 1:
                nxt = [ms[t] | ms[t + 1] for t in range(0, len(ms) - 1, 2)]
                if len(ms) % 2:
                    nxt.append(ms[-1])
                ms = nxt
            return acc | jnp.where(ms[0], 1, 0)
        anyv = lax.fori_loop(0, NGRP, scan_group, zi)
        anyn = jnp.max(plsc.all_reduce_population_count(anyv > 0))

        @pl.when(anyn > 0)
        def _slow():
            pltpu.sync_copy(dst_hbm.at[pl.ds(ebase + ci * CHUNK, CHUNK)], dstb)

            def gbody(g, _):
                b0 = g * (GU * 16)
                m16 = zi
                for t in range(GU):
                    v = buf[pl.ds(b0 + t * 16, 16)]
                    m16 = m16 | jnp.where((v == iv) | (v == jv), 1, 0)
                gn = jnp.max(plsc.all_reduce_population_count(m16 > 0))

                @pl.when(gn > 0)
                def _ghit():
                    for t in range(GU):
                        v = buf[pl.ds(b0 + t * 16, 16)]
                        ei = v == iv
                        ej = v == jv
                        m = jnp.max(plsc.all_reduce_population_count(ei | ej))

                        @pl.when(m > 0)
                        def _hit(t=t, ei=ei, ej=ej):
                            vd = dstb[pl.ds(b0 + t * 16, 16)]
                            idx_i = jnp.where(ei, vd, dump_i)
                            idx_j = jnp.where(ej, vd + R, dump_j)
                            pltpu.sync_copy(onesb, mask_sh.at[idx_i], add=True)
                            pltpu.sync_copy(onesb, mask_sh.at[idx_j], add=True)
                return 0
            lax.fori_loop(0, NGRP, gbody, 0)

    def outer(o, _):
        for b in range(NBUF):
            ci = o * NBUF + b
            scan_chunk(ci, bufs[b], sems[b])

            @pl.when(ci + NBUF < NCHUNK)
            def _next(ci=ci, b=b):
                pltpu.async_copy(chunk_ref(ci + NBUF), bufs[b], sems[b])
        return 0
    lax.fori_loop(0, NCHUNK // NBUF, outer, 0)

    plsc.subcore_barrier()
    off = sid * SLICE1
    pltpu.sync_copy(mask_sh.at[pl.ds(off, SLICE1)],
                    masks_out.at[pl.ds(cid * 2 * R + off, SLICE1)])


def _phase2(masks_hbm, emb_hbm, tl_hbm, partials_out, product_out,
            mi0b, mi1b, mj0b, mj1b, hitb, accb, rowb, tlb, ijrows, prodb, sem):
    cid = lax.axis_index("c")
    sid = lax.axis_index("s")
    wid = cid * NS + sid

    iota = lax.iota(jnp.int32, 16)
    base = wid * NODES_PER_W

    pltpu.sync_copy(masks_hbm.at[pl.ds(base, NODES_PER_W)], mi0b)
    pltpu.sync_copy(masks_hbm.at[pl.ds(2 * R + base, NODES_PER_W)], mi1b)
    pltpu.sync_copy(masks_hbm.at[pl.ds(R + base, NODES_PER_W)], mj0b)
    pltpu.sync_copy(masks_hbm.at[pl.ds(3 * R + base, NODES_PER_W)], mj1b)

    zf = jnp.zeros((16,), jnp.float32)
    for v in range(D // 16):
        accb[pl.ds(v * 16, 16)] = zf

    def common_at(k):
        s = pl.ds(k * 16, 16)
        mi = mi0b[s] + mi1b[s]
        mj = mj0b[s] + mj1b[s]
        u = base + k * 16 + iota
        return (mi > 0.0) & (mj > 0.0) & (u < N_NODES), u

    def qbody(k, acc):
        c, _ = common_at(k)
        return acc | jnp.where(c, 1, 0)
    anyv = lax.fori_loop(0, VECS2, qbody, jnp.zeros((16,), jnp.int32))
    anyn = jnp.max(plsc.all_reduce_population_count(anyv > 0))

    @pl.when(anyn > 0)
    def _collect():
        zi = jnp.zeros((16,), jnp.int32)

        def pbody(k, _):
            hitb[pl.ds(k * 16, 16)] = zi
            return 0
        lax.fori_loop(0, (NODES_PER_W + 16) // 16, pbody, 0)

        def cbody(k, off):
            c, u = common_at(k)
            plsc.store_compressed(hitb.at[pl.ds(off, 16)], u, mask=c)
            return off + jnp.max(plsc.all_reduce_population_count(c))
        nh = lax.fori_loop(0, VECS2, cbody, 0)

        ngr = (nh + 7) // 8

        def gbody(g, _):
            pltpu.sync_copy(emb_hbm.at[hitb.at[pl.ds(g * 8, 8)]], rowb)
            for r in range(8):
                for v in range(D // 16):
                    s = pl.ds(v * 16, 16)
                    accb[s] = accb[s] + rowb[r, s]
            return 0
        lax.fori_loop(0, ngr, gbody, 0)

        pad = (ngr * 8 - nh).astype(jnp.float32)

        @pl.when(pad > 0.0)
        def _unpad():
            padv = jnp.broadcast_to(pad, (16,))
            for v in range(D // 16):
                s = pl.ds(v * 16, 16)
                accb[s] = accb[s] - padv * rowb[7, s]

    pltpu.sync_copy(accb, partials_out.at[pl.ds(wid * D, D)])

    @pl.when(wid == 0)
    def _product():
        pltpu.sync_copy(tl_hbm, tlb)
        pltpu.sync_copy(emb_hbm.at[tlb.at[pl.ds(0, 2)]], ijrows)
        for v in range(D // 16):
            s = pl.ds(v * 16, 16)
            prodb[s] = ijrows[0, s] * ijrows[1, s]
        pltpu.sync_copy(prodb, product_out)


_mesh = plsc.VectorSubcoreMesh(core_axis_name="c", subcore_axis_name="s")
_params = pltpu.CompilerParams(needs_layout_passes=False)

_phase1_call = functools.partial(
    pl.kernel,
    out_type=jax.ShapeDtypeStruct((NC * 2 * R,), jnp.float32),
    mesh=_mesh,
    compiler_params=_params,
    scratch_types=[
        pltpu.VMEM((CHUNK,), jnp.int32),
        pltpu.VMEM((CHUNK,), jnp.int32),
        pltpu.VMEM((CHUNK,), jnp.int32),
        pltpu.VMEM((16,), jnp.int32),
        pltpu.VMEM((16,), jnp.int32),
        pltpu.VMEM((16,), jnp.float32),
        pltpu.VMEM((SLICE1,), jnp.float32),
        pltpu.VMEM_SHARED((2 * R,), jnp.float32),
        pltpu.SemaphoreType.DMA,
        pltpu.SemaphoreType.DMA,
    ],
)(_phase1)

_phase2_call = functools.partial(
    pl.kernel,
    out_type=(
        jax.ShapeDtypeStruct((NW * D,), jnp.float32),
        jax.ShapeDtypeStruct((D,), jnp.float32),
    ),
    mesh=_mesh,
    compiler_params=_params,
    scratch_types=[
        pltpu.VMEM((NODES_PER_W,), jnp.float32),
        pltpu.VMEM((NODES_PER_W,), jnp.float32),
        pltpu.VMEM((NODES_PER_W,), jnp.float32),
        pltpu.VMEM((NODES_PER_W,), jnp.float32),
        pltpu.VMEM((NODES_PER_W + 16,), jnp.int32),
        pltpu.VMEM((D,), jnp.float32),
        pltpu.VMEM((8, D), jnp.float32),
        pltpu.VMEM((16,), jnp.int32),
        pltpu.VMEM((2, D), jnp.float32),
        pltpu.VMEM((D,), jnp.float32),
        pltpu.SemaphoreType.DMA,
    ],
)(_phase2)


@jax.jit
def kernel(targetLink, adjacent, NodeEmbedding):
    src = adjacent[0]
    dst = adjacent[1]
    iv = jnp.broadcast_to(targetLink[0], (16,)).astype(jnp.int32)
    jv = jnp.broadcast_to(targetLink[1], (16,)).astype(jnp.int32)
    tl = jnp.zeros((16,), jnp.int32).at[:2].set(targetLink)
    masks = _phase1_call(src, dst, iv, jv)
    partials, product = _phase2_call(masks, NodeEmbedding, tl)
    return jnp.concatenate(
        [product, jnp.sum(partials.reshape(NW, D), axis=0)], axis=0)


# speedup vs baseline: 1.3855x; 1.0011x over previous
import functools

import jax
import jax.numpy as jnp
from jax import lax
from jax.experimental import pallas as pl
from jax.experimental.pallas import tpu as pltpu
from jax.experimental.pallas import tpu_sc as plsc

N_NODES = 100000
N_EDGES = 3200000
D = 128

NC = 2
NS = 16
NW = NC * NS

R = 100352
NODES_PER_W = R // NW
VECS2 = NODES_PER_W // 16
SLICE1 = (2 * R) // NS

EPW = N_EDGES // NW
CHUNK = 10000
VECS1 = CHUNK // 16
NCHUNK = EPW // CHUNK
NBUF = 2
GU = 25
NGRP = VECS1 // GU


def _phase1(src_hbm, dst_hbm, iv_hbm, jv_hbm, masks_out,
            srcb0, srcb1, dstb, ivb, jvb, onesb, zerob, mask_sh, sem0, sem1):
    cid = lax.axis_index("c")
    sid = lax.axis_index("s")
    wid = cid * NS + sid

    iota = lax.iota(jnp.int32, 16)
    ebase = wid * EPW
    bufs = (srcb0, srcb1)
    sems = (sem0, sem1)

    def chunk_ref(ci):
        return src_hbm.at[pl.ds(ebase + ci * CHUNK, CHUNK)]

    pltpu.async_copy(chunk_ref(0), srcb0, sem0)
    pltpu.async_copy(chunk_ref(1), srcb1, sem1)

    pltpu.sync_copy(iv_hbm, ivb)
    pltpu.sync_copy(jv_hbm, jvb)
    onesb[...] = jnp.ones((16,), jnp.float32)
    zf = jnp.zeros((16,), jnp.float32)

    def zbody(k, _):
        zerob[pl.ds(k * 16, 16)] = zf
        return 0
    lax.fori_loop(0, SLICE1 // 16, zbody, 0)
    pltpu.sync_copy(zerob, mask_sh.at[pl.ds(sid * SLICE1, SLICE1)])
    plsc.subcore_barrier()

    iv = ivb[...]
    jv = jvb[...]
    dump_i = N_NODES + iota
    dump_j = R + N_NODES + iota
    zi = jnp.zeros((16,), jnp.int32)

    def scan_chunk(ci, buf, sem):
        pltpu.make_async_copy(chunk_ref(ci), buf, sem).wait()

        def scan_group(g, acc):
            b0 = g * (GU * 16)
            ms = []
            for t in range(GU):
                v = buf[pl.ds(b0 + t * 16, 16)]
                ms.append((v == iv) | (v == jv))
            while len(ms) > 1:
                nxt = [ms[t] | ms[t + 1] for t in range(0, len(ms) - 1, 2)]
                if len(ms) % 2:
                    nxt.append(ms[-1])
                ms = nxt
            return acc | jnp.where(ms[0], 1, 0)
        anyv = lax.fori_loop(0, NGRP, scan_group, zi)
        anyn = jnp.max(plsc.all_reduce_population_count(anyv > 0))

        @pl.when(anyn > 0)
        def _slow():
            pltpu.sync_copy(dst_hbm.at[pl.ds(ebase + ci * CHUNK, CHUNK)], dstb)

            def gbody(g, _):
                b0 = g * (GU * 16)
                m16 = zi
                for t in range(GU):
                    v = buf[pl.ds(b0 + t * 16, 16)]
                    m16 = m16 | jnp.where((v == iv) | (v == jv), 1, 0)
                gn = jnp.max(plsc.all_reduce_population_count(m16 > 0))

                @pl.when(gn > 0)
                def _ghit():
                    for t in range(GU):
                        v = buf[pl.ds(b0 + t * 16, 16)]
                        ei = v == iv
                        ej = v == jv
                        m = jnp.max(plsc.all_reduce_population_count(ei | ej))

                        @pl.when(m > 0)
                        def _hit(t=t, ei=ei, ej=ej):
                            vd = dstb[pl.ds(b0 + t * 16, 16)]
                            idx_i = jnp.where(ei, vd, dump_i)
                            idx_j = jnp.where(ej, vd + R, dump_j)
                            pltpu.sync_copy(onesb, mask_sh.at[idx_i], add=True)
                            pltpu.sync_copy(onesb, mask_sh.at[idx_j], add=True)
                return 0
            lax.fori_loop(0, NGRP, gbody, 0)

    def outer(o, _):
        for b in range(NBUF):
            ci = o * NBUF + b
            scan_chunk(ci, bufs[b], sems[b])

            @pl.when(ci + NBUF < NCHUNK)
            def _next(ci=ci, b=b):
                pltpu.async_copy(chunk_ref(ci + NBUF), bufs[b], sems[b])
        return 0
    lax.fori_loop(0, NCHUNK // NBUF, outer, 0)

    plsc.subcore_barrier()
    off = sid * SLICE1
    pltpu.sync_copy(mask_sh.at[pl.ds(off, SLICE1)],
                    masks_out.at[pl.ds(cid * 2 * R + off, SLICE1)])


def _phase2(masks_hbm, emb_hbm, tl_hbm, partials_out, product_out,
            mi0b, mi1b, mj0b, mj1b, hitb, accb, rowb, tlb, ijrows, prodb, sem):
    cid = lax.axis_index("c")
    sid = lax.axis_index("s")
    wid = cid * NS + sid

    iota = lax.iota(jnp.int32, 16)
    base = wid * NODES_PER_W

    pltpu.sync_copy(masks_hbm.at[pl.ds(base, NODES_PER_W)], mi0b)
    pltpu.sync_copy(masks_hbm.at[pl.ds(2 * R + base, NODES_PER_W)], mi1b)
    pltpu.sync_copy(masks_hbm.at[pl.ds(R + base, NODES_PER_W)], mj0b)
    pltpu.sync_copy(masks_hbm.at[pl.ds(3 * R + base, NODES_PER_W)], mj1b)

    zf = jnp.zeros((16,), jnp.float32)
    for v in range(D // 16):
        accb[pl.ds(v * 16, 16)] = zf

    def common_at(k):
        s = pl.ds(k * 16, 16)
        mi = mi0b[s] + mi1b[s]
        mj = mj0b[s] + mj1b[s]
        u = base + k * 16 + iota
        return (mi > 0.0) & (mj > 0.0) & (u < N_NODES), u

    def qbody(k, acc):
        c, _ = common_at(k)
        return acc | jnp.where(c, 1, 0)
    anyv = lax.fori_loop(0, VECS2, qbody, jnp.zeros((16,), jnp.int32))
    anyn = jnp.max(plsc.all_reduce_population_count(anyv > 0))

    @pl.when(anyn > 0)
    def _collect():
        zi = jnp.zeros((16,), jnp.int32)

        def pbody(k, _):
            hitb[pl.ds(k * 16, 16)] = zi
            return 0
        lax.fori_loop(0, (NODES_PER_W + 16) // 16, pbody, 0)

        def cbody(k, off):
            c, u = common_at(k)
            plsc.store_compressed(hitb.at[pl.ds(off, 16)], u, mask=c)
            return off + jnp.max(plsc.all_reduce_population_count(c))
        nh = lax.fori_loop(0, VECS2, cbody, 0)

        ngr = (nh + 7) // 8

        def gbody(g, _):
            pltpu.sync_copy(emb_hbm.at[hitb.at[pl.ds(g * 8, 8)]], rowb)
            for r in range(8):
                for v in range(D // 16):
                    s = pl.ds(v * 16, 16)
                    accb[s] = accb[s] + rowb[r, s]
            return 0
        lax.fori_loop(0, ngr, gbody, 0)

        pad = (ngr * 8 - nh).astype(jnp.float32)

        @pl.when(pad > 0.0)
        def _unpad():
            padv = jnp.broadcast_to(pad, (16,))
            for v in range(D // 16):
                s = pl.ds(v * 16, 16)
                accb[s] = accb[s] - padv * rowb[7, s]

    pltpu.sync_copy(accb, partials_out.at[pl.ds(wid * D, D)])

    @pl.when(wid == 0)
    def _product():
        pltpu.sync_copy(tl_hbm, tlb)
        pltpu.sync_copy(emb_hbm.at[tlb.at[pl.ds(0, 2)]], ijrows)
        for v in range(D // 16):
            s = pl.ds(v * 16, 16)
            prodb[s] = ijrows[0, s] * ijrows[1, s]
        pltpu.sync_copy(prodb, product_out)


_mesh = plsc.VectorSubcoreMesh(core_axis_name="c", subcore_axis_name="s")
_params = pltpu.CompilerParams(needs_layout_passes=False,
                               skip_device_barrier=True)

_phase1_call = functools.partial(
    pl.kernel,
    out_type=jax.ShapeDtypeStruct((NC * 2 * R,), jnp.float32),
    mesh=_mesh,
    compiler_params=_params,
    scratch_types=[
        pltpu.VMEM((CHUNK,), jnp.int32),
        pltpu.VMEM((CHUNK,), jnp.int32),
        pltpu.VMEM((CHUNK,), jnp.int32),
        pltpu.VMEM((16,), jnp.int32),
        pltpu.VMEM((16,), jnp.int32),
        pltpu.VMEM((16,), jnp.float32),
        pltpu.VMEM((SLICE1,), jnp.float32),
        pltpu.VMEM_SHARED((2 * R,), jnp.float32),
        pltpu.SemaphoreType.DMA,
        pltpu.SemaphoreType.DMA,
    ],
)(_phase1)

_phase2_call = functools.partial(
    pl.kernel,
    out_type=(
        jax.ShapeDtypeStruct((NW * D,), jnp.float32),
        jax.ShapeDtypeStruct((D,), jnp.float32),
    ),
    mesh=_mesh,
    compiler_params=_params,
    scratch_types=[
        pltpu.VMEM((NODES_PER_W,), jnp.float32),
        pltpu.VMEM((NODES_PER_W,), jnp.float32),
        pltpu.VMEM((NODES_PER_W,), jnp.float32),
        pltpu.VMEM((NODES_PER_W,), jnp.float32),
        pltpu.VMEM((NODES_PER_W + 16,), jnp.int32),
        pltpu.VMEM((D,), jnp.float32),
        pltpu.VMEM((8, D), jnp.float32),
        pltpu.VMEM((16,), jnp.int32),
        pltpu.VMEM((2, D), jnp.float32),
        pltpu.VMEM((D,), jnp.float32),
        pltpu.SemaphoreType.DMA,
    ],
)(_phase2)


@jax.jit
def kernel(targetLink, adjacent, NodeEmbedding):
    src = adjacent[0]
    dst = adjacent[1]
    iv = jnp.broadcast_to(targetLink[0], (16,)).astype(jnp.int32)
    jv = jnp.broadcast_to(targetLink[1], (16,)).astype(jnp.int32)
    tl = jnp.zeros((16,), jnp.int32).at[:2].set(targetLink)
    masks = _phase1_call(src, dst, iv, jv)
    partials, product = _phase2_call(masks, NodeEmbedding, tl)
    return jnp.concatenate(
        [product, jnp.sum(partials.reshape(NW, D), axis=0)], axis=0)


# speedup vs baseline: 1.4778x; 1.0666x over previous
import functools

import jax
import jax.numpy as jnp
from jax import lax
from jax.experimental import pallas as pl
from jax.experimental.pallas import tpu as pltpu
from jax.experimental.pallas import tpu_sc as plsc

N_NODES = 100000
N_EDGES = 3200000
D = 128

NC = 2
NS = 16
NW = NC * NS

R = 100352
NODES_PER_W = R // NW
VECS2 = NODES_PER_W // 16
SLICE1 = (2 * R) // NS

EPW = N_EDGES // NW
CHUNK = 10000
VECS1 = CHUNK // 16
NCHUNK = EPW // CHUNK
NBUF = 2
GU = 25
NGRP = VECS1 // GU


def _phase1(edges_hbm, iv_hbm, jv_hbm, masks_out,
            srcb0, srcb1, dstb, ivb, jvb, onesb, zerob, mask_sh, sem0, sem1):
    cid = lax.axis_index("c")
    sid = lax.axis_index("s")
    wid = cid * NS + sid

    iota = lax.iota(jnp.int32, 16)
    ebase = wid * EPW
    bufs = (srcb0, srcb1)
    sems = (sem0, sem1)

    def chunk_ref(ci):
        return edges_hbm.at[pl.ds(ebase + ci * CHUNK, CHUNK)]

    pltpu.async_copy(chunk_ref(0), srcb0, sem0)
    pltpu.async_copy(chunk_ref(1), srcb1, sem1)

    pltpu.sync_copy(iv_hbm, ivb)
    pltpu.sync_copy(jv_hbm, jvb)
    onesb[...] = jnp.ones((16,), jnp.float32)
    zf = jnp.zeros((16,), jnp.float32)

    def zbody(k, _):
        zerob[pl.ds(k * 16, 16)] = zf
        return 0
    lax.fori_loop(0, SLICE1 // 16, zbody, 0)
    pltpu.sync_copy(zerob, mask_sh.at[pl.ds(sid * SLICE1, SLICE1)])
    plsc.subcore_barrier()

    iv = ivb[...]
    jv = jvb[...]
    dump_i = N_NODES + iota
    dump_j = R + N_NODES + iota
    zi = jnp.zeros((16,), jnp.int32)

    def scan_chunk(ci, buf, sem):
        pltpu.make_async_copy(chunk_ref(ci), buf, sem).wait()

        def scan_group(g, acc):
            b0 = g * (GU * 16)
            ms = []
            for t in range(GU):
                v = buf[pl.ds(b0 + t * 16, 16)]
                ms.append((v == iv) | (v == jv))
            while len(ms) > 1:
                nxt = [ms[t] | ms[t + 1] for t in range(0, len(ms) - 1, 2)]
                if len(ms) % 2:
                    nxt.append(ms[-1])
                ms = nxt
            return acc | jnp.where(ms[0], 1, 0)
        anyv = lax.fori_loop(0, NGRP, scan_group, zi)
        anyn = jnp.max(plsc.all_reduce_population_count(anyv > 0))

        @pl.when(anyn > 0)
        def _slow():
            pltpu.sync_copy(
                edges_hbm.at[pl.ds(N_EDGES + ebase + ci * CHUNK, CHUNK)], dstb)

            def gbody(g, _):
                b0 = g * (GU * 16)
                m16 = zi
                for t in range(GU):
                    v = buf[pl.ds(b0 + t * 16, 16)]
                    m16 = m16 | jnp.where((v == iv) | (v == jv), 1, 0)
                gn = jnp.max(plsc.all_reduce_population_count(m16 > 0))

                @pl.when(gn > 0)
                def _ghit():
                    for t in range(GU):
                        v = buf[pl.ds(b0 + t * 16, 16)]
                        ei = v == iv
                        ej = v == jv
                        m = jnp.max(plsc.all_reduce_population_count(ei | ej))

                        @pl.when(m > 0)
                        def _hit(t=t, ei=ei, ej=ej):
                            vd = dstb[pl.ds(b0 + t * 16, 16)]
                            idx_i = jnp.where(ei, vd, dump_i)
                            idx_j = jnp.where(ej, vd + R, dump_j)
                            pltpu.sync_copy(onesb, mask_sh.at[idx_i], add=True)
                            pltpu.sync_copy(onesb, mask_sh.at[idx_j], add=True)
                return 0
            lax.fori_loop(0, NGRP, gbody, 0)

    def outer(o, _):
        for b in range(NBUF):
            ci = o * NBUF + b
            scan_chunk(ci, bufs[b], sems[b])

            @pl.when(ci + NBUF < NCHUNK)
            def _next(ci=ci, b=b):
                pltpu.async_copy(chunk_ref(ci + NBUF), bufs[b], sems[b])
        return 0
    lax.fori_loop(0, NCHUNK // NBUF, outer, 0)

    plsc.subcore_barrier()
    off = sid * SLICE1
    pltpu.sync_copy(mask_sh.at[pl.ds(off, SLICE1)],
                    masks_out.at[pl.ds(cid * 2 * R + off, SLICE1)])


def _phase2(masks_hbm, emb_hbm, tl_hbm, partials_out, product_out,
            mi0b, mi1b, mj0b, mj1b, hitb, accb, rowb, tlb, ijrows, prodb, sem):
    cid = lax.axis_index("c")
    sid = lax.axis_index("s")
    wid = cid * NS + sid

    iota = lax.iota(jnp.int32, 16)
    base = wid * NODES_PER_W

    pltpu.sync_copy(masks_hbm.at[pl.ds(base, NODES_PER_W)], mi0b)
    pltpu.sync_copy(masks_hbm.at[pl.ds(2 * R + base, NODES_PER_W)], mi1b)
    pltpu.sync_copy(masks_hbm.at[pl.ds(R + base, NODES_PER_W)], mj0b)
    pltpu.sync_copy(masks_hbm.at[pl.ds(3 * R + base, NODES_PER_W)], mj1b)

    zf = jnp.zeros((16,), jnp.float32)
    for v in range(D // 16):
        accb[pl.ds(v * 16, 16)] = zf

    def common_at(k):
        s = pl.ds(k * 16, 16)
        mi = mi0b[s] + mi1b[s]
        mj = mj0b[s] + mj1b[s]
        u = base + k * 16 + iota
        return (mi > 0.0) & (mj > 0.0) & (u < N_NODES), u

    def qbody(k, acc):
        c, _ = common_at(k)
        return acc | jnp.where(c, 1, 0)
    anyv = lax.fori_loop(0, VECS2, qbody, jnp.zeros((16,), jnp.int32))
    anyn = jnp.max(plsc.all_reduce_population_count(anyv > 0))

    @pl.when(anyn > 0)
    def _collect():
        zi = jnp.zeros((16,), jnp.int32)

        def pbody(k, _):
            hitb[pl.ds(k * 16, 16)] = zi
            return 0
        lax.fori_loop(0, (NODES_PER_W + 16) // 16, pbody, 0)

        def cbody(k, off):
            c, u = common_at(k)
            plsc.store_compressed(hitb.at[pl.ds(off, 16)], u, mask=c)
            return off + jnp.max(plsc.all_reduce_population_count(c))
        nh = lax.fori_loop(0, VECS2, cbody, 0)

        ngr = (nh + 7) // 8

        def gbody(g, _):
            pltpu.sync_copy(emb_hbm.at[hitb.at[pl.ds(g * 8, 8)]], rowb)
            for r in range(8):
                for v in range(D // 16):
                    s = pl.ds(v * 16, 16)
                    accb[s] = accb[s] + rowb[r, s]
            return 0
        lax.fori_loop(0, ngr, gbody, 0)

        pad = (ngr * 8 - nh).astype(jnp.float32)

        @pl.when(pad > 0.0)
        def _unpad():
            padv = jnp.broadcast_to(pad, (16,))
            for v in range(D // 16):
                s = pl.ds(v * 16, 16)
                accb[s] = accb[s] - padv * rowb[7, s]

    pltpu.sync_copy(accb, partials_out.at[pl.ds(wid * D, D)])

    @pl.when(wid == 0)
    def _product():
        pltpu.sync_copy(tl_hbm, tlb)
        pltpu.sync_copy(emb_hbm.at[tlb.at[pl.ds(0, 2)]], ijrows)
        for v in range(D // 16):
            s = pl.ds(v * 16, 16)
            prodb[s] = ijrows[0, s] * ijrows[1, s]
        pltpu.sync_copy(prodb, product_out)


_mesh = plsc.VectorSubcoreMesh(core_axis_name="c", subcore_axis_name="s")
_params = pltpu.CompilerParams(needs_layout_passes=False,
                               skip_device_barrier=True)

_phase1_call = functools.partial(
    pl.kernel,
    out_type=jax.ShapeDtypeStruct((NC * 2 * R,), jnp.float32),
    mesh=_mesh,
    compiler_params=_params,
    scratch_types=[
        pltpu.VMEM((CHUNK,), jnp.int32),
        pltpu.VMEM((CHUNK,), jnp.int32),
        pltpu.VMEM((CHUNK,), jnp.int32),
        pltpu.VMEM((16,), jnp.int32),
        pltpu.VMEM((16,), jnp.int32),
        pltpu.VMEM((16,), jnp.float32),
        pltpu.VMEM((SLICE1,), jnp.float32),
        pltpu.VMEM_SHARED((2 * R,), jnp.float32),
        pltpu.SemaphoreType.DMA,
        pltpu.SemaphoreType.DMA,
    ],
)(_phase1)

_phase2_call = functools.partial(
    pl.kernel,
    out_type=(
        jax.ShapeDtypeStruct((NW * D,), jnp.float32),
        jax.ShapeDtypeStruct((D,), jnp.float32),
    ),
    mesh=_mesh,
    compiler_params=_params,
    scratch_types=[
        pltpu.VMEM((NODES_PER_W,), jnp.float32),
        pltpu.VMEM((NODES_PER_W,), jnp.float32),
        pltpu.VMEM((NODES_PER_W,), jnp.float32),
        pltpu.VMEM((NODES_PER_W,), jnp.float32),
        pltpu.VMEM((NODES_PER_W + 16,), jnp.int32),
        pltpu.VMEM((D,), jnp.float32),
        pltpu.VMEM((8, D), jnp.float32),
        pltpu.VMEM((16,), jnp.int32),
        pltpu.VMEM((2, D), jnp.float32),
        pltpu.VMEM((D,), jnp.float32),
        pltpu.SemaphoreType.DMA,
    ],
)(_phase2)


@jax.jit
def kernel(targetLink, adjacent, NodeEmbedding):
    edges = adjacent.reshape(2 * N_EDGES)
    iv = jnp.broadcast_to(targetLink[0], (16,)).astype(jnp.int32)
    jv = jnp.broadcast_to(targetLink[1], (16,)).astype(jnp.int32)
    tl = jnp.zeros((16,), jnp.int32).at[:2].set(targetLink)
    masks = _phase1_call(edges, iv, jv)
    partials, product = _phase2_call(masks, NodeEmbedding, tl)
    return jnp.concatenate(
        [product, jnp.sum(partials.reshape(NW, D), axis=0)], axis=0)


# speedup vs baseline: 1.9146x; 1.2955x over previous
import functools

import jax
import jax.numpy as jnp
from jax import lax
from jax.experimental import pallas as pl
from jax.experimental.pallas import tpu as pltpu
from jax.experimental.pallas import tpu_sc as plsc

N_NODES = 100000
N_EDGES = 3200000
D = 128

NC = 2
NS = 16
NW = NC * NS

R = 100352
NODES_PER_W = R // NW
VECS2 = NODES_PER_W // 16
SLICE1 = (2 * R) // NS

CHUNK = 12800
NCG = N_EDGES // CHUNK
LC = (NCG + NW - 1) // NW
VECS1 = CHUNK // 16
NBUF = 2
GU = 25
NGRP = VECS1 // GU


def _phase1(edges_hbm, iv_hbm, jv_hbm, masks_out,
            srcb0, srcb1, ivb, jvb, onesb, zerob, mask_sh, sem0, sem1):
    cid = lax.axis_index("c")
    sid = lax.axis_index("s")
    wid = cid * NS + sid

    iota = lax.iota(jnp.int32, 16)
    bufs = (srcb0, srcb1)
    sems = (sem0, sem1)

    def chunk_ref(ci):
        return edges_hbm.at[:, pl.ds((wid + NW * ci) * CHUNK, CHUNK)]

    pltpu.async_copy(chunk_ref(0), srcb0, sem0)
    pltpu.async_copy(chunk_ref(1), srcb1, sem1)

    pltpu.sync_copy(iv_hbm, ivb)
    pltpu.sync_copy(jv_hbm, jvb)
    onesb[...] = jnp.ones((16,), jnp.float32)
    zf = jnp.zeros((16,), jnp.float32)

    def zbody(k, _):
        zerob[pl.ds(k * 16, 16)] = zf
        return 0
    lax.fori_loop(0, SLICE1 // 16, zbody, 0)
    pltpu.sync_copy(zerob, mask_sh.at[pl.ds(sid * SLICE1, SLICE1)])
    plsc.subcore_barrier()

    iv = ivb[...]
    jv = jvb[...]
    dump_i = N_NODES + iota
    dump_j = R + N_NODES + iota
    zi = jnp.zeros((16,), jnp.int32)

    def scan_chunk(ci, buf, sem):
        pltpu.make_async_copy(chunk_ref(ci), buf, sem).wait()

        def scan_group(g, acc):
            b0 = g * (GU * 16)
            ms = []
            for t in range(GU):
                v = buf[0, pl.ds(b0 + t * 16, 16)]
                ms.append((v == iv) | (v == jv))
            while len(ms) > 1:
                nxt = [ms[t] | ms[t + 1] for t in range(0, len(ms) - 1, 2)]
                if len(ms) % 2:
                    nxt.append(ms[-1])
                ms = nxt
            return acc | jnp.where(ms[0], 1, 0)
        anyv = lax.fori_loop(0, NGRP, scan_group, zi)
        anyn = jnp.max(plsc.all_reduce_population_count(anyv > 0))

        @pl.when(anyn > 0)
        def _slow():
            def gbody(g, _):
                b0 = g * (GU * 16)
                m16 = zi
                for t in range(GU):
                    v = buf[0, pl.ds(b0 + t * 16, 16)]
                    m16 = m16 | jnp.where((v == iv) | (v == jv), 1, 0)
                gn = jnp.max(plsc.all_reduce_population_count(m16 > 0))

                @pl.when(gn > 0)
                def _ghit():
                    for t in range(GU):
                        v = buf[0, pl.ds(b0 + t * 16, 16)]
                        ei = v == iv
                        ej = v == jv
                        m = jnp.max(plsc.all_reduce_population_count(ei | ej))

                        @pl.when(m > 0)
                        def _hit(t=t, ei=ei, ej=ej):
                            vd = buf[1, pl.ds(b0 + t * 16, 16)]
                            idx_i = jnp.where(ei, vd, dump_i)
                            idx_j = jnp.where(ej, vd + R, dump_j)
                            pltpu.sync_copy(onesb, mask_sh.at[idx_i], add=True)
                            pltpu.sync_copy(onesb, mask_sh.at[idx_j], add=True)
                return 0
            lax.fori_loop(0, NGRP, gbody, 0)

    def outer(o, _):
        for b in range(NBUF):
            ci = o * NBUF + b

            @pl.when(wid + NW * ci < NCG)
            def _do(ci=ci, b=b):
                scan_chunk(ci, bufs[b], sems[b])

                @pl.when(wid + NW * (ci + NBUF) < NCG)
                def _next():
                    pltpu.async_copy(chunk_ref(ci + NBUF), bufs[b], sems[b])
        return 0
    lax.fori_loop(0, (LC + NBUF - 1) // NBUF, outer, 0)

    plsc.subcore_barrier()
    off = sid * SLICE1
    pltpu.sync_copy(mask_sh.at[pl.ds(off, SLICE1)],
                    masks_out.at[pl.ds(cid * 2 * R + off, SLICE1)])


def _phase2(masks_hbm, emb_hbm, tl_hbm, partials_out, product_out,
            mi0b, mi1b, mj0b, mj1b, hitb, accb, rowb, tlb, ijrows, prodb, sem):
    cid = lax.axis_index("c")
    sid = lax.axis_index("s")
    wid = cid * NS + sid

    iota = lax.iota(jnp.int32, 16)
    base = wid * NODES_PER_W

    pltpu.sync_copy(masks_hbm.at[pl.ds(base, NODES_PER_W)], mi0b)
    pltpu.sync_copy(masks_hbm.at[pl.ds(2 * R + base, NODES_PER_W)], mi1b)
    pltpu.sync_copy(masks_hbm.at[pl.ds(R + base, NODES_PER_W)], mj0b)
    pltpu.sync_copy(masks_hbm.at[pl.ds(3 * R + base, NODES_PER_W)], mj1b)

    zf = jnp.zeros((16,), jnp.float32)
    for v in range(D // 16):
        accb[pl.ds(v * 16, 16)] = zf

    def common_at(k):
        s = pl.ds(k * 16, 16)
        mi = mi0b[s] + mi1b[s]
        mj = mj0b[s] + mj1b[s]
        u = base + k * 16 + iota
        return (mi > 0.0) & (mj > 0.0) & (u < N_NODES), u

    def qbody(k, acc):
        c, _ = common_at(k)
        return acc | jnp.where(c, 1, 0)
    anyv = lax.fori_loop(0, VECS2, qbody, jnp.zeros((16,), jnp.int32))
    anyn = jnp.max(plsc.all_reduce_population_count(anyv > 0))

    @pl.when(anyn > 0)
    def _collect():
        zi = jnp.zeros((16,), jnp.int32)

        def pbody(k, _):
            hitb[pl.ds(k * 16, 16)] = zi
            return 0
        lax.fori_loop(0, (NODES_PER_W + 16) // 16, pbody, 0)

        def cbody(k, off):
            c, u = common_at(k)
            plsc.store_compressed(hitb.at[pl.ds(off, 16)], u, mask=c)
            return off + jnp.max(plsc.all_reduce_population_count(c))
        nh = lax.fori_loop(0, VECS2, cbody, 0)

        ngr = (nh + 7) // 8

        def gbody(g, _):
            pltpu.sync_copy(emb_hbm.at[hitb.at[pl.ds(g * 8, 8)]], rowb)
            for r in range(8):
                for v in range(D // 16):
                    s = pl.ds(v * 16, 16)
                    accb[s] = accb[s] + rowb[r, s]
            return 0
        lax.fori_loop(0, ngr, gbody, 0)

        pad = (ngr * 8 - nh).astype(jnp.float32)

        @pl.when(pad > 0.0)
        def _unpad():
            padv = jnp.broadcast_to(pad, (16,))
            for v in range(D // 16):
                s = pl.ds(v * 16, 16)
                accb[s] = accb[s] - padv * rowb[7, s]

    pltpu.sync_copy(accb, partials_out.at[pl.ds(wid * D, D)])

    @pl.when(wid == 0)
    def _product():
        pltpu.sync_copy(tl_hbm, tlb)
        pltpu.sync_copy(emb_hbm.at[tlb.at[pl.ds(0, 2)]], ijrows)
        for v in range(D // 16):
            s = pl.ds(v * 16, 16)
            prodb[s] = ijrows[0, s] * ijrows[1, s]
        pltpu.sync_copy(prodb, product_out)


_mesh = plsc.VectorSubcoreMesh(core_axis_name="c", subcore_axis_name="s")
_params = pltpu.CompilerParams(needs_layout_passes=False,
                               skip_device_barrier=True)

_phase1_call = functools.partial(
    pl.kernel,
    out_type=jax.ShapeDtypeStruct((NC * 2 * R,), jnp.float32),
    mesh=_mesh,
    compiler_params=_params,
    scratch_types=[
        pltpu.VMEM((2, CHUNK), jnp.int32),
        pltpu.VMEM((2, CHUNK), jnp.int32),
        pltpu.VMEM((16,), jnp.int32),
        pltpu.VMEM((16,), jnp.int32),
        pltpu.VMEM((16,), jnp.float32),
        pltpu.VMEM((SLICE1,), jnp.float32),
        pltpu.VMEM_SHARED((2 * R,), jnp.float32),
        pltpu.SemaphoreType.DMA,
        pltpu.SemaphoreType.DMA,
    ],
)(_phase1)

_phase2_call = functools.partial(
    pl.kernel,
    out_type=(
        jax.ShapeDtypeStruct((NW * D,), jnp.float32),
        jax.ShapeDtypeStruct((D,), jnp.float32),
    ),
    mesh=_mesh,
    compiler_params=_params,
    scratch_types=[
        pltpu.VMEM((NODES_PER_W,), jnp.float32),
        pltpu.VMEM((NODES_PER_W,), jnp.float32),
        pltpu.VMEM((NODES_PER_W,), jnp.float32),
        pltpu.VMEM((NODES_PER_W,), jnp.float32),
        pltpu.VMEM((NODES_PER_W + 16,), jnp.int32),
        pltpu.VMEM((D,), jnp.float32),
        pltpu.VMEM((8, D), jnp.float32),
        pltpu.VMEM((16,), jnp.int32),
        pltpu.VMEM((2, D), jnp.float32),
        pltpu.VMEM((D,), jnp.float32),
        pltpu.SemaphoreType.DMA,
    ],
)(_phase2)


@jax.jit
def kernel(targetLink, adjacent, NodeEmbedding):
    iv = jnp.broadcast_to(targetLink[0], (16,)).astype(jnp.int32)
    jv = jnp.broadcast_to(targetLink[1], (16,)).astype(jnp.int32)
    tl = jnp.zeros((16,), jnp.int32).at[:2].set(targetLink)
    masks = _phase1_call(adjacent, iv, jv)
    partials, product = _phase2_call(masks, NodeEmbedding, tl)
    return jnp.concatenate(
        [product, jnp.sum(partials.reshape(NW, D), axis=0)], axis=0)


# speedup vs baseline: 2.1341x; 1.1147x over previous
import functools

import jax
import jax.numpy as jnp
from jax import lax
from jax.experimental import pallas as pl
from jax.experimental.pallas import tpu as pltpu
from jax.experimental.pallas import tpu_sc as plsc

N_NODES = 100000
N_EDGES = 3200000
D = 128

NC = 2
NS = 16
NW = NC * NS

R = 100352
NODES_PER_W = R // NW
VECS2 = NODES_PER_W // 16
SLICE1 = (2 * R) // NS

CHUNK = 6400
NCG = N_EDGES // CHUNK
LC = (NCG + NW - 1) // NW
VECS1 = CHUNK // 16
NBUF = 2
GU = 25
NGRP = VECS1 // GU


def _phase1(edges_hbm, tl_hbm, masks_out,
            srcb0, srcb1, tlb, onesb, zerob, mask_sh, sem0, sem1):
    cid = lax.axis_index("c")
    sid = lax.axis_index("s")
    wid = cid * NS + sid

    iota = lax.iota(jnp.int32, 16)
    bufs = (srcb0, srcb1)
    sems = (sem0, sem1)

    def chunk_ref(ci):
        return edges_hbm.at[:, pl.ds((wid + NW * ci) * CHUNK, CHUNK)]

    pltpu.async_copy(chunk_ref(0), srcb0, sem0)
    pltpu.async_copy(chunk_ref(1), srcb1, sem1)

    pltpu.sync_copy(tl_hbm, tlb)
    onesb[...] = jnp.ones((16,), jnp.float32)
    zf = jnp.zeros((16,), jnp.float32)

    def zbody(k, _):
        zerob[pl.ds(k * 16, 16)] = zf
        return 0
    lax.fori_loop(0, SLICE1 // 16, zbody, 0)
    pltpu.sync_copy(zerob, mask_sh.at[pl.ds(sid * SLICE1, SLICE1)])
    plsc.subcore_barrier()

    tlv = tlb[...]
    iv = jnp.broadcast_to(tlv[0], (16,))
    jv = jnp.broadcast_to(tlv[1], (16,))
    dump_i = N_NODES + iota
    dump_j = R + N_NODES + iota
    zi = jnp.zeros((16,), jnp.int32)

    def scan_chunk(ci, buf, sem):
        pltpu.make_async_copy(chunk_ref(ci), buf, sem).wait()

        def scan_group(g, acc):
            b0 = g * (GU * 16)
            ms = []
            for t in range(GU):
                v = buf[0, pl.ds(b0 + t * 16, 16)]
                ms.append((v == iv) | (v == jv))
            while len(ms) > 1:
                nxt = [ms[t] | ms[t + 1] for t in range(0, len(ms) - 1, 2)]
                if len(ms) % 2:
                    nxt.append(ms[-1])
                ms = nxt
            return acc | jnp.where(ms[0], 1, 0)
        anyv = lax.fori_loop(0, NGRP, scan_group, zi)
        anyn = jnp.max(plsc.all_reduce_population_count(anyv > 0))

        @pl.when(anyn > 0)
        def _slow():
            def gbody(g, _):
                b0 = g * (GU * 16)
                m16 = zi
                for t in range(GU):
                    v = buf[0, pl.ds(b0 + t * 16, 16)]
                    m16 = m16 | jnp.where((v == iv) | (v == jv), 1, 0)
                gn = jnp.max(plsc.all_reduce_population_count(m16 > 0))

                @pl.when(gn > 0)
                def _ghit():
                    for t in range(GU):
                        v = buf[0, pl.ds(b0 + t * 16, 16)]
                        ei = v == iv
                        ej = v == jv
                        m = jnp.max(plsc.all_reduce_population_count(ei | ej))

                        @pl.when(m > 0)
                        def _hit(t=t, ei=ei, ej=ej):
                            vd = buf[1, pl.ds(b0 + t * 16, 16)]
                            idx_i = jnp.where(ei, vd, dump_i)
                            idx_j = jnp.where(ej, vd + R, dump_j)
                            pltpu.sync_copy(onesb, mask_sh.at[idx_i], add=True)
                            pltpu.sync_copy(onesb, mask_sh.at[idx_j], add=True)
                return 0
            lax.fori_loop(0, NGRP, gbody, 0)

    def outer(o, _):
        for b in range(NBUF):
            ci = o * NBUF + b

            @pl.when(wid + NW * ci < NCG)
            def _do(ci=ci, b=b):
                scan_chunk(ci, bufs[b], sems[b])

                @pl.when(wid + NW * (ci + NBUF) < NCG)
                def _next():
                    pltpu.async_copy(chunk_ref(ci + NBUF), bufs[b], sems[b])
        return 0
    lax.fori_loop(0, (LC + NBUF - 1) // NBUF, outer, 0)

    plsc.subcore_barrier()
    off = sid * SLICE1
    pltpu.sync_copy(mask_sh.at[pl.ds(off, SLICE1)],
                    masks_out.at[pl.ds(cid * 2 * R + off, SLICE1)])


def _phase2(masks_hbm, emb_hbm, tl_hbm, partials_out, product_out,
            mi0b, mi1b, mj0b, mj1b, hitb, accb, rowb, tlb, ijrows, prodb, sem):
    cid = lax.axis_index("c")
    sid = lax.axis_index("s")
    wid = cid * NS + sid

    iota = lax.iota(jnp.int32, 16)
    base = wid * NODES_PER_W

    pltpu.sync_copy(masks_hbm.at[pl.ds(base, NODES_PER_W)], mi0b)
    pltpu.sync_copy(masks_hbm.at[pl.ds(2 * R + base, NODES_PER_W)], mi1b)
    pltpu.sync_copy(masks_hbm.at[pl.ds(R + base, NODES_PER_W)], mj0b)
    pltpu.sync_copy(masks_hbm.at[pl.ds(3 * R + base, NODES_PER_W)], mj1b)

    zf = jnp.zeros((16,), jnp.float32)
    for v in range(D // 16):
        accb[pl.ds(v * 16, 16)] = zf

    def common_at(k):
        s = pl.ds(k * 16, 16)
        mi = mi0b[s] + mi1b[s]
        mj = mj0b[s] + mj1b[s]
        u = base + k * 16 + iota
        return (mi > 0.0) & (mj > 0.0) & (u < N_NODES), u

    def qbody(k, acc):
        c, _ = common_at(k)
        return acc | jnp.where(c, 1, 0)
    anyv = lax.fori_loop(0, VECS2, qbody, jnp.zeros((16,), jnp.int32))
    anyn = jnp.max(plsc.all_reduce_population_count(anyv > 0))

    @pl.when(anyn > 0)
    def _collect():
        zi = jnp.zeros((16,), jnp.int32)

        def pbody(k, _):
            hitb[pl.ds(k * 16, 16)] = zi
            return 0
        lax.fori_loop(0, (NODES_PER_W + 16) // 16, pbody, 0)

        def cbody(k, off):
            c, u = common_at(k)
            plsc.store_compressed(hitb.at[pl.ds(off, 16)], u, mask=c)
            return off + jnp.max(plsc.all_reduce_population_count(c))
        nh = lax.fori_loop(0, VECS2, cbody, 0)

        ngr = (nh + 7) // 8

        def gbody(g, _):
            pltpu.sync_copy(emb_hbm.at[hitb.at[pl.ds(g * 8, 8)]], rowb)
            for r in range(8):
                for v in range(D // 16):
                    s = pl.ds(v * 16, 16)
                    accb[s] = accb[s] + rowb[r, s]
            return 0
        lax.fori_loop(0, ngr, gbody, 0)

        pad = (ngr * 8 - nh).astype(jnp.float32)

        @pl.when(pad > 0.0)
        def _unpad():
            padv = jnp.broadcast_to(pad, (16,))
            for v in range(D // 16):
                s = pl.ds(v * 16, 16)
                accb[s] = accb[s] - padv * rowb[7, s]

    pltpu.sync_copy(accb, partials_out.at[pl.ds(wid * D, D)])

    @pl.when(wid == 0)
    def _product():
        pltpu.sync_copy(tl_hbm, tlb)
        pltpu.sync_copy(emb_hbm.at[tlb.at[pl.ds(0, 2)]], ijrows)
        for v in range(D // 16):
            s = pl.ds(v * 16, 16)
            prodb[s] = ijrows[0, s] * ijrows[1, s]
        pltpu.sync_copy(prodb, product_out)


_mesh = plsc.VectorSubcoreMesh(core_axis_name="c", subcore_axis_name="s")
_params = pltpu.CompilerParams(needs_layout_passes=False,
                               skip_device_barrier=True)

_phase1_call = functools.partial(
    pl.kernel,
    out_type=jax.ShapeDtypeStruct((NC * 2 * R,), jnp.float32),
    mesh=_mesh,
    compiler_params=_params,
    scratch_types=[
        pltpu.VMEM((2, CHUNK), jnp.int32),
        pltpu.VMEM((2, CHUNK), jnp.int32),
        pltpu.VMEM((16,), jnp.int32),
        pltpu.VMEM((16,), jnp.float32),
        pltpu.VMEM((SLICE1,), jnp.float32),
        pltpu.VMEM_SHARED((2 * R,), jnp.float32),
        pltpu.SemaphoreType.DMA,
        pltpu.SemaphoreType.DMA,
    ],
)(_phase1)

_phase2_call = functools.partial(
    pl.kernel,
    out_type=(
        jax.ShapeDtypeStruct((NW * D,), jnp.float32),
        jax.ShapeDtypeStruct((D,), jnp.float32),
    ),
    mesh=_mesh,
    compiler_params=_params,
    scratch_types=[
        pltpu.VMEM((NODES_PER_W,), jnp.float32),
        pltpu.VMEM((NODES_PER_W,), jnp.float32),
        pltpu.VMEM((NODES_PER_W,), jnp.float32),
        pltpu.VMEM((NODES_PER_W,), jnp.float32),
        pltpu.VMEM((NODES_PER_W + 16,), jnp.int32),
        pltpu.VMEM((D,), jnp.float32),
        pltpu.VMEM((8, D), jnp.float32),
        pltpu.VMEM((16,), jnp.int32),
        pltpu.VMEM((2, D), jnp.float32),
        pltpu.VMEM((D,), jnp.float32),
        pltpu.SemaphoreType.DMA,
    ],
)(_phase2)


@jax.jit
def kernel(targetLink, adjacent, NodeEmbedding):
    tl = jnp.zeros((16,), jnp.int32).at[:2].set(targetLink)
    masks = _phase1_call(adjacent, tl)
    partials, product = _phase2_call(masks, NodeEmbedding, tl)
    return jnp.concatenate(
        [product, jnp.sum(partials.reshape(NW, D), axis=0)], axis=0)
